# baseline (speedup 1.0000x reference)
import functools

import jax
import jax.numpy as jnp
from jax import lax
from jax.experimental import pallas as pl
from jax.experimental.pallas import tpu as pltpu
from jax.experimental.pallas import tpu_sc as plsc

N = 50000
E = 800000
CLIN = 64
MEL = 128
HID = 64
HEADS = 2
NCLS = 4

NC = 2
NS = 16
NW = NC * NS

BN = 1000
T1W = 80
ACCW = 144
T2W = 16
NPAD = 50176
CK = 6272
CKP = CK + 128
NPH = 4
G = 128
NCHUNK = E // G

_i32 = jnp.int32
_f32 = jnp.float32


def _elu(x):
    return jnp.where(x > 0, x, jnp.exp(jnp.minimum(x, 0.0)) - 1.0)


def _lrelu(x):
    return jnp.where(x > 0, x, 0.2 * x)


def _ab_body(clin_ref, mel_ref, Wm_ref, bm_ref, Wct_ref, Wcb_ref, bc_ref,
             W1_ref, asv_ref, adv_ref, T1_ref, D1_ref):
    m = jnp.maximum(mel_ref[...] @ Wm_ref[...] + bm_ref[...], 0.0)
    pre = clin_ref[...] @ Wct_ref[...] + m @ Wcb_ref[...] + bc_ref[...]
    x = _elu(pre)
    h1 = x @ W1_ref[...]
    p = h1 * asv_ref[...]
    q = h1 * adv_ref[...]
    a1s0 = jnp.sum(p[:, :HID], axis=1, keepdims=True)
    a1s1 = jnp.sum(p[:, HID:], axis=1, keepdims=True)
    a1d0 = jnp.sum(q[:, :HID], axis=1, keepdims=True)
    a1d1 = jnp.sum(q[:, HID:], axis=1, keepdims=True)
    z14 = jnp.zeros((BN, 14), _f32)
    T1_ref[...] = jnp.concatenate([x, a1s0, a1s1, z14], axis=1)
    D1_ref[...] = jnp.concatenate([a1d0, a1d1], axis=1)


def _tc_ab(clinical, mel, Wm, bm, Wct, Wcb, bc, W1, asv, adv):
    return pl.pallas_call(
        _ab_body,
        grid=(N // BN,),
        in_specs=[
            pl.BlockSpec((BN, CLIN), lambda i: (i, 0)),
            pl.BlockSpec((BN, MEL), lambda i: (i, 0)),
            pl.BlockSpec((MEL, HID), lambda i: (0, 0)),
            pl.BlockSpec((HID,), lambda i: (0,)),
            pl.BlockSpec((CLIN, HID), lambda i: (0, 0)),
            pl.BlockSpec((HID, HID), lambda i: (0, 0)),
            pl.BlockSpec((HID,), lambda i: (0,)),
            pl.BlockSpec((HID, HEADS * HID), lambda i: (0, 0)),
            pl.BlockSpec((1, HEADS * HID), lambda i: (0, 0)),
            pl.BlockSpec((1, HEADS * HID), lambda i: (0, 0)),
        ],
        out_specs=[
            pl.BlockSpec((BN, T1W), lambda i: (i, 0)),
            pl.BlockSpec((BN, 2), lambda i: (i, 0)),
        ],
        out_shape=[
            jax.ShapeDtypeStruct((N, T1W), _f32),
            jax.ShapeDtypeStruct((N, 2), _f32),
        ],
    )(clinical, mel, Wm, bm, Wct, Wcb, bc, W1, asv, adv)


def _c_body(acc_ref, T1_ref, D1_ref, W1_ref, W2_ref, as2_ref, ad2_ref, b1_ref,
            T2_ref):
    x = T1_ref[:, :HID]
    a1s = T1_ref[:, HID:HID + 2]
    a1d = D1_ref[...]
    ws = jnp.exp(_lrelu(a1s + a1d))
    aggx0 = acc_ref[:, :HID] + x * ws[:, 0:1]
    aggx1 = acc_ref[:, HID:2 * HID] + x * ws[:, 1:2]
    den = acc_ref[:, 2 * HID:2 * HID + 2] + ws
    m0 = aggx0 @ W1_ref[:, :HID]
    m1 = aggx1 @ W1_ref[:, HID:]
    out1 = jnp.concatenate([
        m0 / (jnp.broadcast_to(den[:, 0:1], (BN, HID)) + 1e-16),
        m1 / (jnp.broadcast_to(den[:, 1:2], (BN, HID)) + 1e-16),
    ], axis=1) + b1_ref[...]
    x2 = _elu(out1)
    h2 = x2 @ W2_ref[...]
    a2s = jnp.sum(h2 * as2_ref[...], axis=1, keepdims=True)
    a2d = jnp.sum(h2 * ad2_ref[...], axis=1, keepdims=True)
    z10 = jnp.zeros((BN, 10), _f32)
    T2_ref[...] = jnp.concatenate([h2, a2s, a2d, z10], axis=1)


def _tc_c(acc1, T1, D1, W1, W2, as2v, ad2v, b1):
    return pl.pallas_call(
        _c_body,
        grid=(N // BN,),
        in_specs=[
            pl.BlockSpec((BN, ACCW), lambda i: (i, 0)),
            pl.BlockSpec((BN, T1W), lambda i: (i, 0)),
            pl.BlockSpec((BN, 2), lambda i: (i, 0)),
            pl.BlockSpec((HID, HEADS * HID), lambda i: (0, 0)),
            pl.BlockSpec((HEADS * HID, NCLS), lambda i: (0, 0)),
            pl.BlockSpec((1, NCLS), lambda i: (0, 0)),
            pl.BlockSpec((1, NCLS), lambda i: (0, 0)),
            pl.BlockSpec((HEADS * HID,), lambda i: (0,)),
        ],
        out_specs=pl.BlockSpec((BN, T2W), lambda i: (i, 0)),
        out_shape=jax.ShapeDtypeStruct((N, T2W), _f32),
    )(acc1, T1, D1, W1, W2, as2v, ad2v, b1)


def _e_body(acc2_ref, T2_ref, b2_ref, out_ref):
    asum = acc2_ref[0] + acc2_ref[1]
    num = asum[:, :NCLS]
    den = asum[:, NCLS:NCLS + 1]
    h2 = T2_ref[:, :NCLS]
    a2s = T2_ref[:, NCLS:NCLS + 1]
    a2d = T2_ref[:, NCLS + 1:NCLS + 2]
    ws = jnp.exp(_lrelu(a2s + a2d))
    num = num + h2 * ws
    den = den + ws
    out_ref[...] = num / (den + 1e-16) + b2_ref[...]


def _tc_e(acc2, T2, b2):
    return pl.pallas_call(
        _e_body,
        grid=(N // BN,),
        in_specs=[
            pl.BlockSpec((2, BN, T2W), lambda i: (0, i, 0)),
            pl.BlockSpec((BN, T2W), lambda i: (i, 0)),
            pl.BlockSpec((NCLS,), lambda i: (0,)),
        ],
        out_specs=pl.BlockSpec((BN, NCLS), lambda i: (i, 0)),
        out_shape=jax.ShapeDtypeStruct((N, NCLS), _f32),
    )(acc2, T2, b2)


_MESH = plsc.VectorSubcoreMesh(core_axis_name="c", subcore_axis_name="s",
                               num_cores=NC, num_subcores=NS)
_SC_PARAMS = pltpu.CompilerParams(needs_layout_passes=False,
                                  use_tc_tiling_on_sc=False)


def _iota16():
    return lax.iota(_i32, 16)


def _zero_rows(buf, nrows, ncols):
    z = jnp.zeros((16,), _f32)

    def body(i, _):
        for c in range(ncols // 16):
            buf[i, pl.ds(c * 16, 16)] = z
        return 0

    lax.fori_loop(0, nrows, body, 0)


def _copy_rows(src_at, dst_at, total):
    done = 0
    while done < total:
        step = min(G, total - done)
        pltpu.sync_copy(src_at(done, step), dst_at(done, step))
        done += step


def _p1_kernel(ei_hbm, t1_hbm, d1_hbm, acc_hbm,
               ebA, ebB, pend, pbuf, sidx, idxl,
               gbuf, a1dloc, msg, wbuf0, wbuf1, accsh,
               esemA, esemB, gsem1):
    c = lax.axis_index("c")
    s = lax.axis_index("s")
    iota = _iota16()

    def issue_edges(eb, sem, i):
        pltpu.async_copy(ei_hbm.at[:, pl.ds(i * G, G)], eb, sem)

    def wait_edges(eb, sem):
        pltpu.make_async_copy(ei_hbm.at[:, pl.ds(0, G)], eb, sem).wait()

    def stage_and_issue(kbase):
        for v in range(8):
            pv = pend[pl.ds(v * 16, 16)]
            sv = pv & 0xFFFF
            dv = lax.shift_right_logical(pv, 16)
            sidx[pl.ds(v * 16, 16)] = sv
            idxl[0, pl.ds(v * 16, 16)] = dv - kbase
        for v in range(9):
            pend[pl.ds(v * 16, 16)] = pend[pl.ds(128 + v * 16, 16)]
        pltpu.async_copy(t1_hbm.at[sidx], gbuf, gsem1)

    def complete_flush():
        pltpu.make_async_copy(t1_hbm.at[sidx], gbuf, gsem1).wait()

        def ggroup(g, carry):
            rows = g * 16 + iota
            a1s0 = plsc.load_gather(gbuf, [rows, jnp.full((16,), HID, _i32)])
            a1s1 = plsc.load_gather(gbuf, [rows, jnp.full((16,), HID + 1, _i32)])
            dl2 = idxl[0, pl.ds(g * 16, 16)] * 2
            a1d0 = plsc.load_gather(a1dloc, [dl2])
            a1d1 = plsc.load_gather(a1dloc, [dl2 + 1])
            w0 = jnp.exp(_lrelu(a1s0 + a1d0))
            w1 = jnp.exp(_lrelu(a1s1 + a1d1))
            wbuf0[pl.ds(0, 16)] = w0
            wbuf1[pl.ds(0, 16)] = w1
            for r in range(16):
                er = g * 16 + r
                rfull = jnp.full((16,), r, _i32)
                w0r = plsc.load_gather(wbuf0, [rfull])
                w1r = plsc.load_gather(wbuf1, [rfull])
                dv = jnp.where(iota == 0, w0r,
                               jnp.where(iota == 1, w1r, 0.0))
                for cc in range(4):
                    xv = gbuf[er, pl.ds(cc * 16, 16)]
                    msg[er, pl.ds(cc * 16, 16)] = xv * w0r
                    msg[er, pl.ds(64 + cc * 16, 16)] = xv * w1r
                msg[er, pl.ds(128, 16)] = dv
            return carry

        lax.fori_loop(0, 8, ggroup, 0)
        pltpu.sync_copy(msg, accsh.at[idxl.at[0]], add=True)

    def scan_chunk(eb, cnt, nfl, kbase):
        for g in range(8):
            sv = eb[0, pl.ds(g * 16, 16)]
            dv = eb[1, pl.ds(g * 16, 16)]
            dl = dv - kbase
            m = (dl >= 0) & (dl < CK)
            pv = jnp.bitwise_or(sv, lax.shift_left(dv, 16))
            x = jnp.where(m, 1, 0)
            incl = x
            for sh in (1, 2, 4, 8):
                pbuf[pl.ds(0, 16)] = incl
                shifted = plsc.load_gather(pbuf, [jnp.maximum(iota - sh, 0)])
                incl = incl + jnp.where(iota >= sh, shifted, 0)
            pos = cnt + incl - x
            plsc.store_scatter(pend, [pos], pv, mask=m)
            cnt = cnt + plsc.all_reduce_population_count(m)

        def do_flush(args):
            cn, f = args
            pl.when(f == 1)(complete_flush)
            stage_and_issue(kbase)
            return cn - 128, jnp.int32(1)

        return lax.cond(jnp.any(cnt >= 128), do_flush, lambda a: a, (cnt, nfl))

    def phase_body(p, _):
        k = c * NPH + p
        kbase = k * CK

        pltpu.sync_copy(d1_hbm.at[pl.ds(kbase * 2, CK * 2)],
                        a1dloc.at[pl.ds(0, CK * 2)])
        z16 = jnp.zeros((16,), _f32)
        a1dloc[pl.ds(CK * 2, 16)] = z16
        a1dloc[pl.ds(CK * 2 + 16, 16)] = z16

        _zero_rows(msg, G, ACCW)
        zrows = CKP // NS
        zbase = s * zrows
        _copy_rows(lambda o, n: msg.at[pl.ds(0, n)],
                   lambda o, n: accsh.at[pl.ds(zbase + o, n)], zrows)
        plsc.subcore_barrier()

        nb = 390 + jnp.where(s < 10, 1, 0)
        nb2 = (nb + 1) // 2
        issue_edges(ebA, esemA, s)
        issue_edges(ebB, esemB, s + NS)

        def body2(j2, carry):
            cnt, nfl = carry
            jA = 2 * j2
            jB = jA + 1
            wait_edges(ebA, esemA)
            cnt, nfl = scan_chunk(ebA, cnt, nfl, kbase)
            pl.when(jA + 2 < nb)(
                lambda: issue_edges(ebA, esemA, s + (jA + 2) * NS))

            def procB(args):
                cn, f = args
                wait_edges(ebB, esemB)
                return scan_chunk(ebB, cn, f, kbase)

            cnt, nfl = lax.cond(jB < nb, procB, lambda a: a, (cnt, nfl))
            pl.when(jB + 2 < nb)(
                lambda: issue_edges(ebB, esemB, s + (jB + 2) * NS))
            return cnt, nfl

        cnt, nfl = lax.fori_loop(
            0, nb2, body2, (jnp.zeros((16,), _i32), jnp.int32(0)))
        pl.when(nfl == 1)(complete_flush)

        dummy = lax.shift_left(jnp.broadcast_to(kbase + CK, (16,)), 16)
        for v in range(8):
            lid = v * 16 + iota
            valid = lid < cnt
            pv = jnp.where(valid, pend[pl.ds(v * 16, 16)], dummy)
            pend[pl.ds(v * 16, 16)] = pv
        stage_and_issue(kbase)
        complete_flush()
        plsc.subcore_barrier()

        drows = CK // NS
        dbase = s * drows
        _copy_rows(lambda o, n: accsh.at[pl.ds(dbase + o, n)],
                   lambda o, n: acc_hbm.at[pl.ds(kbase + dbase + o, n)], drows)
        plsc.subcore_barrier()
        return 0

    lax.fori_loop(0, NPH, phase_body, 0)


def _sc_p1(ei, T1, D1):
    f = pl.kernel(
        _p1_kernel,
        out_type=jax.ShapeDtypeStruct((NPAD, ACCW), _f32),
        mesh=_MESH,
        compiler_params=_SC_PARAMS,
        scratch_types=[
            pltpu.VMEM((2, G), _i32),
            pltpu.VMEM((2, G), _i32),
            pltpu.VMEM((272,), _i32),
            pltpu.VMEM((16,), _i32),
            pltpu.VMEM((G,), _i32),
            pltpu.VMEM((1, G), _i32),
            pltpu.VMEM((G, T1W), _f32),
            pltpu.VMEM((2 * (CK + 16),), _f32),
            pltpu.VMEM((G, ACCW), _f32),
            pltpu.VMEM((16,), _f32),
            pltpu.VMEM((16,), _f32),
            pltpu.VMEM_SHARED((CKP, ACCW), _f32),
            pltpu.SemaphoreType.DMA,
            pltpu.SemaphoreType.DMA,
            pltpu.SemaphoreType.DMA,
        ],
    )
    return f(ei, T1, D1)


def _p2_kernel(ei_hbm, t2_hbm, acc_hbm,
               ebA, ebB, gsA, gdA, gsB, gdB, wbuf, msg, accsh,
               esemA, esemB, gssemA, gdsemA, gssemB, gdsemB):
    c = lax.axis_index("c")
    s = lax.axis_index("s")
    w = s * NC + c
    iota = _iota16()

    _zero_rows(msg, G, T2W)
    zrows = NPAD // NS
    zbase = s * zrows
    _copy_rows(lambda o, n: msg.at[pl.ds(0, n)],
               lambda o, n: accsh.at[pl.ds(zbase + o, n)], zrows)
    plsc.subcore_barrier()

    def issue_edges(eb, sem, i):
        pltpu.async_copy(ei_hbm.at[:, pl.ds(i * G, G)], eb, sem)

    def wait_edges(eb, sem):
        pltpu.make_async_copy(ei_hbm.at[:, pl.ds(0, G)], eb, sem).wait()

    def issue_g(eb, gs, gd, ssem, dsem):
        pltpu.async_copy(t2_hbm.at[eb.at[0]], gs, ssem)
        pltpu.async_copy(t2_hbm.at[eb.at[1]], gd, dsem)

    def wait_g(eb, gs, gd, ssem, dsem):
        pltpu.make_async_copy(t2_hbm.at[eb.at[0]], gs, ssem).wait()
        pltpu.make_async_copy(t2_hbm.at[eb.at[1]], gd, dsem).wait()

    def compute_scatter(eb, gs, gd):
        def ggroup(g, cg):
            rows = g * 16 + iota
            a2s = plsc.load_gather(gs, [rows, jnp.full((16,), 4, _i32)])
            a2d = plsc.load_gather(gd, [rows, jnp.full((16,), 5, _i32)])
            wv = jnp.exp(_lrelu(a2s + a2d))
            wbuf[pl.ds(0, 16)] = wv
            for r in range(16):
                er = g * 16 + r
                wr = plsc.load_gather(wbuf, [jnp.full((16,), r, _i32)])
                row = gs[er, pl.ds(0, 16)]
                scaled = row * wr
                msg[er, pl.ds(0, 16)] = jnp.where(
                    iota < NCLS, scaled, jnp.where(iota == NCLS, wr, 0.0))
            return cg

        lax.fori_loop(0, 8, ggroup, 0)
        pltpu.sync_copy(msg, accsh.at[eb.at[1]], add=True)

    nb = 195 + jnp.where(w < 10, 1, 0)
    nb2 = (nb + 1) // 2
    issue_edges(ebA, esemA, w)
    issue_edges(ebB, esemB, w + NW)
    wait_edges(ebA, esemA)
    issue_g(ebA, gsA, gdA, gssemA, gdsemA)

    def body2(j2, carry):
        jA = 2 * j2
        jB = jA + 1

        def startB():
            wait_edges(ebB, esemB)
            issue_g(ebB, gsB, gdB, gssemB, gdsemB)

        pl.when(jB < nb)(startB)
        wait_g(ebA, gsA, gdA, gssemA, gdsemA)
        compute_scatter(ebA, gsA, gdA)
        pl.when(jA + 2 < nb)(
            lambda: issue_edges(ebA, esemA, w + (jA + 2) * NW))

        def finishB():
            wait_g(ebB, gsB, gdB, gssemB, gdsemB)
            compute_scatter(ebB, gsB, gdB)

        pl.when(jB < nb)(finishB)
        pl.when(jB + 2 < nb)(
            lambda: issue_edges(ebB, esemB, w + (jB + 2) * NW))

        def nextA():
            wait_edges(ebA, esemA)
            issue_g(ebA, gsA, gdA, gssemA, gdsemA)

        pl.when(jA + 2 < nb)(nextA)
        return carry

    lax.fori_loop(0, nb2, body2, 0)
    plsc.subcore_barrier()

    drows = NPAD // NS
    dbase = s * drows
    _copy_rows(lambda o, n: accsh.at[pl.ds(dbase + o, n)],
               lambda o, n: acc_hbm.at[c, pl.ds(dbase + o, n)], drows)


def _sc_p2(ei, T2):
    f = pl.kernel(
        _p2_kernel,
        out_type=jax.ShapeDtypeStruct((NC, NPAD, T2W), _f32),
        mesh=_MESH,
        compiler_params=_SC_PARAMS,
        scratch_types=[
            pltpu.VMEM((2, G), _i32),
            pltpu.VMEM((2, G), _i32),
            pltpu.VMEM((G, T2W), _f32),
            pltpu.VMEM((G, T2W), _f32),
            pltpu.VMEM((G, T2W), _f32),
            pltpu.VMEM((G, T2W), _f32),
            pltpu.VMEM((16,), _f32),
            pltpu.VMEM((G, T2W), _f32),
            pltpu.VMEM_SHARED((NPAD, T2W), _f32),
            pltpu.SemaphoreType.DMA,
            pltpu.SemaphoreType.DMA,
            pltpu.SemaphoreType.DMA,
            pltpu.SemaphoreType.DMA,
            pltpu.SemaphoreType.DMA,
            pltpu.SemaphoreType.DMA,
        ],
    )
    return f(ei, T2)


def kernel(clinical, mel, edge_index, Wm, bm, Wc, bc, W1, as1, ad1, b1, W2, as2, ad2, b2):
    asv1 = as1.reshape(1, HEADS * HID)
    adv1 = ad1.reshape(1, HEADS * HID)
    as2v = as2.reshape(1, NCLS)
    ad2v = ad2.reshape(1, NCLS)
    Wct = Wc[:CLIN]
    Wcb = Wc[CLIN:]
    T1, D1 = _tc_ab(clinical, mel, Wm, bm, Wct, Wcb, bc, W1, asv1, adv1)
    acc1 = _sc_p1(edge_index, T1, D1.reshape(2 * N))
    T2 = _tc_c(acc1, T1, D1, W1, W2, as2v, ad2v, b1)
    acc2 = _sc_p2(edge_index, T2)
    return _tc_e(acc2, T2, b2)


# speedup vs baseline: 1.0701x; 1.0701x over previous
import functools

import jax
import jax.numpy as jnp
from jax import lax
from jax.experimental import pallas as pl
from jax.experimental.pallas import tpu as pltpu
from jax.experimental.pallas import tpu_sc as plsc

N = 50000
E = 800000
CLIN = 64
MEL = 128
HID = 64
HEADS = 2
NCLS = 4

NC = 2
NS = 16
NW = NC * NS

BN = 1000
T1W = 80
ACCW = 144
T2W = 16
NPAD = 50688
CK = 8448
CKP = CK + 128
NPH = 3
G = 128
NCHUNK = E // G

_i32 = jnp.int32
_f32 = jnp.float32


def _elu(x):
    return jnp.where(x > 0, x, jnp.exp(jnp.minimum(x, 0.0)) - 1.0)


def _lrelu(x):
    return jnp.where(x > 0, x, 0.2 * x)


def _ab_body(clin_ref, mel_ref, Wm_ref, bm_ref, Wct_ref, Wcb_ref, bc_ref,
             W1_ref, asv_ref, adv_ref, T1_ref, D1_ref):
    m = jnp.maximum(mel_ref[...] @ Wm_ref[...] + bm_ref[...], 0.0)
    pre = clin_ref[...] @ Wct_ref[...] + m @ Wcb_ref[...] + bc_ref[...]
    x = _elu(pre)
    h1 = x @ W1_ref[...]
    p = h1 * asv_ref[...]
    q = h1 * adv_ref[...]
    a1s0 = jnp.sum(p[:, :HID], axis=1, keepdims=True)
    a1s1 = jnp.sum(p[:, HID:], axis=1, keepdims=True)
    a1d0 = jnp.sum(q[:, :HID], axis=1, keepdims=True)
    a1d1 = jnp.sum(q[:, HID:], axis=1, keepdims=True)
    z14 = jnp.zeros((BN, 14), _f32)
    T1_ref[...] = jnp.concatenate([x, a1s0, a1s1, z14], axis=1)
    D1_ref[...] = jnp.concatenate([a1d0, a1d1], axis=1)


def _tc_ab(clinical, mel, Wm, bm, Wct, Wcb, bc, W1, asv, adv):
    return pl.pallas_call(
        _ab_body,
        grid=(N // BN,),
        in_specs=[
            pl.BlockSpec((BN, CLIN), lambda i: (i, 0)),
            pl.BlockSpec((BN, MEL), lambda i: (i, 0)),
            pl.BlockSpec((MEL, HID), lambda i: (0, 0)),
            pl.BlockSpec((HID,), lambda i: (0,)),
            pl.BlockSpec((CLIN, HID), lambda i: (0, 0)),
            pl.BlockSpec((HID, HID), lambda i: (0, 0)),
            pl.BlockSpec((HID,), lambda i: (0,)),
            pl.BlockSpec((HID, HEADS * HID), lambda i: (0, 0)),
            pl.BlockSpec((1, HEADS * HID), lambda i: (0, 0)),
            pl.BlockSpec((1, HEADS * HID), lambda i: (0, 0)),
        ],
        out_specs=[
            pl.BlockSpec((BN, T1W), lambda i: (i, 0)),
            pl.BlockSpec((BN, 2), lambda i: (i, 0)),
        ],
        out_shape=[
            jax.ShapeDtypeStruct((N, T1W), _f32),
            jax.ShapeDtypeStruct((N, 2), _f32),
        ],
    )(clinical, mel, Wm, bm, Wct, Wcb, bc, W1, asv, adv)


def _c_body(acc_ref, T1_ref, D1_ref, W1_ref, W2_ref, as2_ref, ad2_ref, b1_ref,
            T2_ref):
    x = T1_ref[:, :HID]
    a1s = T1_ref[:, HID:HID + 2]
    a1d = D1_ref[...]
    ws = jnp.exp(_lrelu(a1s + a1d))
    aggx0 = acc_ref[:, :HID] + x * ws[:, 0:1]
    aggx1 = acc_ref[:, HID:2 * HID] + x * ws[:, 1:2]
    den = acc_ref[:, 2 * HID:2 * HID + 2] + ws
    m0 = aggx0 @ W1_ref[:, :HID]
    m1 = aggx1 @ W1_ref[:, HID:]
    out1 = jnp.concatenate([
        m0 / (jnp.broadcast_to(den[:, 0:1], (BN, HID)) + 1e-16),
        m1 / (jnp.broadcast_to(den[:, 1:2], (BN, HID)) + 1e-16),
    ], axis=1) + b1_ref[...]
    x2 = _elu(out1)
    h2 = x2 @ W2_ref[...]
    a2s = jnp.sum(h2 * as2_ref[...], axis=1, keepdims=True)
    a2d = jnp.sum(h2 * ad2_ref[...], axis=1, keepdims=True)
    z10 = jnp.zeros((BN, 10), _f32)
    T2_ref[...] = jnp.concatenate([h2, a2s, a2d, z10], axis=1)


def _tc_c(acc1, T1, D1, W1, W2, as2v, ad2v, b1):
    return pl.pallas_call(
        _c_body,
        grid=(N // BN,),
        in_specs=[
            pl.BlockSpec((BN, ACCW), lambda i: (i, 0)),
            pl.BlockSpec((BN, T1W), lambda i: (i, 0)),
            pl.BlockSpec((BN, 2), lambda i: (i, 0)),
            pl.BlockSpec((HID, HEADS * HID), lambda i: (0, 0)),
            pl.BlockSpec((HEADS * HID, NCLS), lambda i: (0, 0)),
            pl.BlockSpec((1, NCLS), lambda i: (0, 0)),
            pl.BlockSpec((1, NCLS), lambda i: (0, 0)),
            pl.BlockSpec((HEADS * HID,), lambda i: (0,)),
        ],
        out_specs=pl.BlockSpec((BN, T2W), lambda i: (i, 0)),
        out_shape=jax.ShapeDtypeStruct((N, T2W), _f32),
    )(acc1, T1, D1, W1, W2, as2v, ad2v, b1)


def _e_body(acc2_ref, T2_ref, b2_ref, out_ref):
    asum = acc2_ref[0] + acc2_ref[1]
    num = asum[:, :NCLS]
    den = asum[:, NCLS:NCLS + 1]
    h2 = T2_ref[:, :NCLS]
    a2s = T2_ref[:, NCLS:NCLS + 1]
    a2d = T2_ref[:, NCLS + 1:NCLS + 2]
    ws = jnp.exp(_lrelu(a2s + a2d))
    num = num + h2 * ws
    den = den + ws
    out_ref[...] = num / (den + 1e-16) + b2_ref[...]


def _tc_e(acc2, T2, b2):
    return pl.pallas_call(
        _e_body,
        grid=(N // BN,),
        in_specs=[
            pl.BlockSpec((2, BN, T2W), lambda i: (0, i, 0)),
            pl.BlockSpec((BN, T2W), lambda i: (i, 0)),
            pl.BlockSpec((NCLS,), lambda i: (0,)),
        ],
        out_specs=pl.BlockSpec((BN, NCLS), lambda i: (i, 0)),
        out_shape=jax.ShapeDtypeStruct((N, NCLS), _f32),
    )(acc2, T2, b2)


_MESH = plsc.VectorSubcoreMesh(core_axis_name="c", subcore_axis_name="s",
                               num_cores=NC, num_subcores=NS)
_SC_PARAMS = pltpu.CompilerParams(needs_layout_passes=False,
                                  use_tc_tiling_on_sc=False)


def _iota16():
    return lax.iota(_i32, 16)


def _zero_rows(buf, nrows, ncols):
    z = jnp.zeros((16,), _f32)

    def body(i, _):
        for c in range(ncols // 16):
            buf[i, pl.ds(c * 16, 16)] = z
        return 0

    lax.fori_loop(0, nrows, body, 0)


def _copy_rows(src_at, dst_at, total):
    done = 0
    while done < total:
        step = min(G, total - done)
        pltpu.sync_copy(src_at(done, step), dst_at(done, step))
        done += step


def _p1_kernel(ei_hbm, t1_hbm, d1_hbm, acc_hbm,
               ebA, ebB, pend, pbuf, sidx, idxl,
               gbuf, a1dloc, msg, wbuf0, wbuf1, accsh,
               esemA, esemB, gsem1):
    c = lax.axis_index("c")
    s = lax.axis_index("s")
    iota = _iota16()

    def issue_edges(eb, sem, i):
        pltpu.async_copy(ei_hbm.at[:, pl.ds(i * G, G)], eb, sem)

    def wait_edges(eb, sem):
        pltpu.make_async_copy(ei_hbm.at[:, pl.ds(0, G)], eb, sem).wait()

    def stage_and_issue(kbase):
        def unpack(v, _):
            pv = pend[pl.ds(v * 16, 16)]
            sv = pv & 0xFFFF
            dv = lax.shift_right_logical(pv, 16)
            sidx[pl.ds(v * 16, 16)] = sv
            idxl[0, pl.ds(v * 16, 16)] = dv - kbase
            return 0

        lax.fori_loop(0, 8, unpack, 0)

        def shift(v, _):
            pend[pl.ds(v * 16, 16)] = pend[pl.ds(128 + v * 16, 16)]
            return 0

        lax.fori_loop(0, 9, shift, 0)
        pltpu.async_copy(t1_hbm.at[sidx], gbuf, gsem1)

    def complete_flush():
        pltpu.make_async_copy(t1_hbm.at[sidx], gbuf, gsem1).wait()

        def ggroup(g, carry):
            rows = g * 16 + iota
            a1s0 = plsc.load_gather(gbuf, [rows, jnp.full((16,), HID, _i32)])
            a1s1 = plsc.load_gather(gbuf, [rows, jnp.full((16,), HID + 1, _i32)])
            dl2 = idxl[0, pl.ds(g * 16, 16)] * 2
            a1d0 = plsc.load_gather(a1dloc, [dl2])
            a1d1 = plsc.load_gather(a1dloc, [dl2 + 1])
            w0 = jnp.exp(_lrelu(a1s0 + a1d0))
            w1 = jnp.exp(_lrelu(a1s1 + a1d1))
            wbuf0[pl.ds(0, 16)] = w0
            wbuf1[pl.ds(0, 16)] = w1
            for r in range(16):
                er = g * 16 + r
                rfull = jnp.full((16,), r, _i32)
                w0r = plsc.load_gather(wbuf0, [rfull])
                w1r = plsc.load_gather(wbuf1, [rfull])
                dv = jnp.where(iota == 0, w0r,
                               jnp.where(iota == 1, w1r, 0.0))
                for cc in range(4):
                    xv = gbuf[er, pl.ds(cc * 16, 16)]
                    msg[er, pl.ds(cc * 16, 16)] = xv * w0r
                    msg[er, pl.ds(64 + cc * 16, 16)] = xv * w1r
                msg[er, pl.ds(128, 16)] = dv
            return carry

        lax.fori_loop(0, 8, ggroup, 0)
        pltpu.sync_copy(msg, accsh.at[idxl.at[0]], add=True)

    def scan_chunk(eb, cnt, nfl, kbase):
        def sgroup(g, cn):
            sv = eb[0, pl.ds(g * 16, 16)]
            dv = eb[1, pl.ds(g * 16, 16)]
            dl = dv - kbase
            m = (dl >= 0) & (dl < CK)
            pv = jnp.bitwise_or(sv, lax.shift_left(dv, 16))
            x = jnp.where(m, 1, 0)
            incl = x
            for sh in (1, 2, 4, 8):
                pbuf[pl.ds(0, 16)] = incl
                shifted = plsc.load_gather(pbuf, [jnp.maximum(iota - sh, 0)])
                incl = incl + jnp.where(iota >= sh, shifted, 0)
            pos = cn + incl - x
            plsc.store_scatter(pend, [pos], pv, mask=m)
            return cn + plsc.all_reduce_population_count(m)

        cnt = lax.fori_loop(0, 8, sgroup, cnt)

        def do_flush(args):
            cn, f = args
            pl.when(f == 1)(complete_flush)
            stage_and_issue(kbase)
            return cn - 128, jnp.int32(1)

        return lax.cond(jnp.any(cnt >= 128), do_flush, lambda a: a, (cnt, nfl))

    for p in range(NPH):
        k = c * NPH + p
        kbase = k * CK

        pltpu.sync_copy(d1_hbm.at[pl.ds(kbase * 2, CK * 2)],
                        a1dloc.at[pl.ds(0, CK * 2)])
        z16 = jnp.zeros((16,), _f32)
        a1dloc[pl.ds(CK * 2, 16)] = z16
        a1dloc[pl.ds(CK * 2 + 16, 16)] = z16

        _zero_rows(msg, G, ACCW)
        zrows = CKP // NS
        zbase = s * zrows
        _copy_rows(lambda o, n: msg.at[pl.ds(0, n)],
                   lambda o, n: accsh.at[pl.ds(zbase + o, n)], zrows)
        plsc.subcore_barrier()

        nb = 390 + jnp.where(s < 10, 1, 0)
        nb2 = (nb + 1) // 2
        issue_edges(ebA, esemA, s)
        issue_edges(ebB, esemB, s + NS)

        def body2(j2, carry):
            cnt, nfl = carry
            jA = 2 * j2
            jB = jA + 1
            wait_edges(ebA, esemA)
            cnt, nfl = scan_chunk(ebA, cnt, nfl, kbase)
            pl.when(jA + 2 < nb)(
                lambda: issue_edges(ebA, esemA, s + (jA + 2) * NS))

            def procB(args):
                cn, f = args
                wait_edges(ebB, esemB)
                return scan_chunk(ebB, cn, f, kbase)

            cnt, nfl = lax.cond(jB < nb, procB, lambda a: a, (cnt, nfl))
            pl.when(jB + 2 < nb)(
                lambda: issue_edges(ebB, esemB, s + (jB + 2) * NS))
            return cnt, nfl

        cnt, nfl = lax.fori_loop(
            0, nb2, body2, (jnp.zeros((16,), _i32), jnp.int32(0)))
        pl.when(nfl == 1)(complete_flush)

        dummy = lax.shift_left(jnp.broadcast_to(kbase + CK, (16,)), 16)
        for v in range(8):
            lid = v * 16 + iota
            valid = lid < cnt
            pv = jnp.where(valid, pend[pl.ds(v * 16, 16)], dummy)
            pend[pl.ds(v * 16, 16)] = pv
        stage_and_issue(kbase)
        complete_flush()
        plsc.subcore_barrier()

        drows = CK // NS
        dbase = s * drows
        _copy_rows(lambda o, n: accsh.at[pl.ds(dbase + o, n)],
                   lambda o, n: acc_hbm.at[pl.ds(kbase + dbase + o, n)], drows)
        plsc.subcore_barrier()


def _sc_p1(ei, T1, D1):
    f = pl.kernel(
        _p1_kernel,
        out_type=jax.ShapeDtypeStruct((NPAD, ACCW), _f32),
        mesh=_MESH,
        compiler_params=_SC_PARAMS,
        scratch_types=[
            pltpu.VMEM((2, G), _i32),
            pltpu.VMEM((2, G), _i32),
            pltpu.VMEM((272,), _i32),
            pltpu.VMEM((16,), _i32),
            pltpu.VMEM((G,), _i32),
            pltpu.VMEM((1, G), _i32),
            pltpu.VMEM((G, T1W), _f32),
            pltpu.VMEM((2 * (CK + 16),), _f32),
            pltpu.VMEM((G, ACCW), _f32),
            pltpu.VMEM((16,), _f32),
            pltpu.VMEM((16,), _f32),
            pltpu.VMEM_SHARED((CKP, ACCW), _f32),
            pltpu.SemaphoreType.DMA,
            pltpu.SemaphoreType.DMA,
            pltpu.SemaphoreType.DMA,
        ],
    )
    return f(ei, T1, D1)


def _p2_kernel(ei_hbm, t2_hbm, acc_hbm,
               ebA, ebB, gsA, gdA, gsB, gdB, wbuf, msg, accsh,
               esemA, esemB, gssemA, gdsemA, gssemB, gdsemB):
    c = lax.axis_index("c")
    s = lax.axis_index("s")
    w = s * NC + c
    iota = _iota16()

    _zero_rows(msg, G, T2W)
    zrows = NPAD // NS
    zbase = s * zrows
    _copy_rows(lambda o, n: msg.at[pl.ds(0, n)],
               lambda o, n: accsh.at[pl.ds(zbase + o, n)], zrows)
    plsc.subcore_barrier()

    def issue_edges(eb, sem, i):
        pltpu.async_copy(ei_hbm.at[:, pl.ds(i * G, G)], eb, sem)

    def wait_edges(eb, sem):
        pltpu.make_async_copy(ei_hbm.at[:, pl.ds(0, G)], eb, sem).wait()

    def issue_g(eb, gs, gd, ssem, dsem):
        pltpu.async_copy(t2_hbm.at[eb.at[0]], gs, ssem)
        pltpu.async_copy(t2_hbm.at[eb.at[1]], gd, dsem)

    def wait_g(eb, gs, gd, ssem, dsem):
        pltpu.make_async_copy(t2_hbm.at[eb.at[0]], gs, ssem).wait()
        pltpu.make_async_copy(t2_hbm.at[eb.at[1]], gd, dsem).wait()

    def compute_scatter(eb, gs, gd):
        def ggroup(g, cg):
            rows = g * 16 + iota
            a2s = plsc.load_gather(gs, [rows, jnp.full((16,), 4, _i32)])
            a2d = plsc.load_gather(gd, [rows, jnp.full((16,), 5, _i32)])
            wv = jnp.exp(_lrelu(a2s + a2d))
            wbuf[pl.ds(0, 16)] = wv
            for r in range(16):
                er = g * 16 + r
                wr = plsc.load_gather(wbuf, [jnp.full((16,), r, _i32)])
                row = gs[er, pl.ds(0, 16)]
                scaled = row * wr
                msg[er, pl.ds(0, 16)] = jnp.where(
                    iota < NCLS, scaled, jnp.where(iota == NCLS, wr, 0.0))
            return cg

        lax.fori_loop(0, 8, ggroup, 0)
        pltpu.sync_copy(msg, accsh.at[eb.at[1]], add=True)

    nb = 195 + jnp.where(w < 10, 1, 0)
    nb2 = (nb + 1) // 2
    issue_edges(ebA, esemA, w)
    issue_edges(ebB, esemB, w + NW)
    wait_edges(ebA, esemA)
    issue_g(ebA, gsA, gdA, gssemA, gdsemA)

    def body2(j2, carry):
        jA = 2 * j2
        jB = jA + 1

        def startB():
            wait_edges(ebB, esemB)
            issue_g(ebB, gsB, gdB, gssemB, gdsemB)

        pl.when(jB < nb)(startB)
        wait_g(ebA, gsA, gdA, gssemA, gdsemA)
        compute_scatter(ebA, gsA, gdA)
        pl.when(jA + 2 < nb)(
            lambda: issue_edges(ebA, esemA, w + (jA + 2) * NW))

        def finishB():
            wait_g(ebB, gsB, gdB, gssemB, gdsemB)
            compute_scatter(ebB, gsB, gdB)

        pl.when(jB < nb)(finishB)
        pl.when(jB + 2 < nb)(
            lambda: issue_edges(ebB, esemB, w + (jB + 2) * NW))

        def nextA():
            wait_edges(ebA, esemA)
            issue_g(ebA, gsA, gdA, gssemA, gdsemA)

        pl.when(jA + 2 < nb)(nextA)
        return carry

    lax.fori_loop(0, nb2, body2, 0)
    plsc.subcore_barrier()

    drows = NPAD // NS
    dbase = s * drows
    _copy_rows(lambda o, n: accsh.at[pl.ds(dbase + o, n)],
               lambda o, n: acc_hbm.at[c, pl.ds(dbase + o, n)], drows)


def _sc_p2(ei, T2):
    f = pl.kernel(
        _p2_kernel,
        out_type=jax.ShapeDtypeStruct((NC, NPAD, T2W), _f32),
        mesh=_MESH,
        compiler_params=_SC_PARAMS,
        scratch_types=[
            pltpu.VMEM((2, G), _i32),
            pltpu.VMEM((2, G), _i32),
            pltpu.VMEM((G, T2W), _f32),
            pltpu.VMEM((G, T2W), _f32),
            pltpu.VMEM((G, T2W), _f32),
            pltpu.VMEM((G, T2W), _f32),
            pltpu.VMEM((16,), _f32),
            pltpu.VMEM((G, T2W), _f32),
            pltpu.VMEM_SHARED((NPAD, T2W), _f32),
            pltpu.SemaphoreType.DMA,
            pltpu.SemaphoreType.DMA,
            pltpu.SemaphoreType.DMA,
            pltpu.SemaphoreType.DMA,
            pltpu.SemaphoreType.DMA,
            pltpu.SemaphoreType.DMA,
        ],
    )
    return f(ei, T2)


def kernel(clinical, mel, edge_index, Wm, bm, Wc, bc, W1, as1, ad1, b1, W2, as2, ad2, b2):
    asv1 = as1.reshape(1, HEADS * HID)
    adv1 = ad1.reshape(1, HEADS * HID)
    as2v = as2.reshape(1, NCLS)
    ad2v = ad2.reshape(1, NCLS)
    Wct = Wc[:CLIN]
    Wcb = Wc[CLIN:]
    T1, D1 = _tc_ab(clinical, mel, Wm, bm, Wct, Wcb, bc, W1, asv1, adv1)
    acc1 = _sc_p1(edge_index, T1, D1.reshape(2 * N))
    T2 = _tc_c(acc1, T1, D1, W1, W2, as2v, ad2v, b1)
    acc2 = _sc_p2(edge_index, T2)
    return _tc_e(acc2, T2, b2)
